# speedup vs baseline: 56.2596x; 56.2596x over previous
import functools

import jax
import jax.numpy as jnp
from jax.experimental import pallas as pl
from jax.experimental.pallas import tpu as pltpu

_B, _N, _D = 16384, 7, 256
_C = 64


def _table_kernel(rank_ref, suit_ref, card_ref, w_ref, m_ref):
    row = jax.lax.broadcasted_iota(jnp.int32, (_C, 1), 0)
    valid = row < 52
    ranks = row % 13
    suits = row // 13
    oh_r = jnp.where(
        (ranks == jax.lax.broadcasted_iota(jnp.int32, (_C, 16), 1)) & valid,
        1.0, 0.0)
    oh_s = jnp.where(
        (suits == jax.lax.broadcasted_iota(jnp.int32, (_C, 8), 1)) & valid,
        1.0, 0.0)
    t = (
        jax.lax.dot_general(oh_r, rank_ref[...],
                            (((1,), (0,)), ((), ())),
                            preferred_element_type=jnp.float32)
        + jax.lax.dot_general(oh_s, suit_ref[...],
                              (((1,), (0,)), ((), ())),
                              preferred_element_type=jnp.float32)
        + card_ref[...]
    )
    m_ref[...] = jax.lax.dot_general(
        t, w_ref[...], (((1,), (1,)), ((), ())),
        preferred_element_type=jnp.float32)


def _build_table(rank_emb, suit_emb, card_emb, W):
    rank_pad = jnp.zeros((16, _D), jnp.float32).at[:13].set(rank_emb)
    suit_pad = jnp.zeros((8, _D), jnp.float32).at[:4].set(suit_emb)
    card_pad = jnp.zeros((_C, _D), jnp.float32).at[:52].set(card_emb)
    return pl.pallas_call(
        _table_kernel,
        out_shape=jax.ShapeDtypeStruct((_C, _D), jnp.float32),
    )(rank_pad, suit_pad, card_pad, W)


def _main_kernel(cards_ref, m_ref, b_ref, out_ref):
    cards = cards_ref[...]
    bins = jax.lax.broadcasted_iota(jnp.int32, (cards.shape[0], _C), 1)
    counts = jnp.zeros((cards.shape[0], _C), jnp.float32)
    for n in range(_N):
        counts += jnp.where(cards[:, n:n + 1] == bins, 1.0, 0.0)
    acc = jax.lax.dot_general(
        counts, m_ref[...], (((1,), (0,)), ((), ())),
        preferred_element_type=jnp.float32)
    out_ref[...] = jnp.maximum(acc + b_ref[...], 0.0)


def kernel(cards, rank_emb, suit_emb, card_emb, W, b):
    m = _build_table(rank_emb, suit_emb, card_emb, W)
    blk = 2048
    grid = (_B // blk,)
    return pl.pallas_call(
        _main_kernel,
        grid=grid,
        in_specs=[
            pl.BlockSpec((blk, _N), lambda i: (i, 0)),
            pl.BlockSpec((_C, _D), lambda i: (0, 0)),
            pl.BlockSpec((1, _D), lambda i: (0, 0)),
        ],
        out_specs=pl.BlockSpec((blk, _D), lambda i: (i, 0)),
        out_shape=jax.ShapeDtypeStruct((_B, _D), jnp.float32),
    )(cards, m, b.reshape(1, _D))
